# baseline (speedup 1.0000x reference)
import functools

import jax
import jax.numpy as jnp
from jax import lax
from jax.experimental import pallas as pl
from jax.experimental.pallas import tpu as pltpu
from jax.experimental.pallas import tpu_sc as plsc

N_NODES = 10000
N_EDGES = 320000
F_IN = 128
HEADS = 8
C1 = 128
C2 = 64
NUM_CLASS = 40
SLOPE = 0.1
EPS = 1e-16

NC = 2
NS = 16
NW = NC * NS
BB = 40

_f32 = jnp.float32
_i32 = jnp.int32


def _mo8(v):
    return pl.multiple_of(v, 8)


_GDN = lax.GatherDimensionNumbers(offset_dims=(), collapsed_slice_dims=(0,),
                                  start_index_map=(0,))


def _bcast_lane(vec, lane):
    idx = jnp.full((16, 1), lane, _i32)
    return lax.gather(vec, idx, _GDN, (1,),
                      mode=lax.GatherScatterMode.PROMISE_IN_BOUNDS)


def _row_split(n):
    main = ((n + NS - 1) // NS + 7) // 8 * 8
    last = n - (NS - 1) * main
    assert last > 0 and last % 8 == 0
    return main, last


def _tile_copy(src_fn, dst_fn, sid, main, last):
    @pl.when(sid < NS - 1)
    def _():
        pltpu.sync_copy(src_fn(main), dst_fn(main))

    @pl.when(sid == NS - 1)
    def _():
        pltpu.sync_copy(src_fn(last), dst_fn(last))


def _make_tc1(n, f_in, heads, ch, blk):
    d = heads * ch

    nt = d // 128

    def body(x_ref, w_ref, ap_ref, *out_refs):
        h = jnp.dot(x_ref[...], w_ref[...], preferred_element_type=_f32)
        for i in range(nt):
            out_refs[i][...] = h[:, i * 128:(i + 1) * 128]
        out_refs[nt][...] = jnp.dot(h, ap_ref[...], preferred_element_type=_f32)

    grid = (n // blk,)
    in_specs = [
        pl.BlockSpec((blk, f_in), lambda i: (i, 0)),
        pl.BlockSpec((f_in, d), lambda i: (0, 0)),
        pl.BlockSpec((d, 128), lambda i: (0, 0)),
    ]
    out_specs = [pl.BlockSpec((blk, 128), lambda i: (i, 0)) for _ in range(nt + 1)]
    out_shape = [jax.ShapeDtypeStruct((n, 128), _f32) for _ in range(nt + 1)]
    return pl.pallas_call(body, grid=grid, in_specs=in_specs,
                          out_specs=out_specs, out_shape=out_shape)


def _combine(p, dn, b, h, ch):
    hpt = 128 // ch
    t, off = h // hpt, (h % hpt) * ch
    rec = 1.0 / (dn[0] + dn[1] + EPS)
    ph = p[0, t][:, off:off + ch] + p[1, t][:, off:off + ch]
    xh = ph * rec[:, h][:, None] + b[h][None, :]
    return jnp.maximum(xh, 0.0)


def _make_tc2(n, heads, ch_in, ch_out, blk):
    d_out = heads * ch_out
    nt_in = heads * ch_in // 128
    nt_out = d_out // 128

    def body(p_ref, dn_ref, b_ref, w_ref, ap_ref, *out_refs):
        p = p_ref[...]
        dn = dn_ref[...]
        b = b_ref[...]
        acc = jnp.zeros((blk, d_out), _f32)
        for h in range(heads):
            xh = _combine(p, dn, b, h, ch_in)
            acc = acc + jnp.dot(xh, w_ref[h], preferred_element_type=_f32)
        for i in range(nt_out):
            out_refs[i][...] = acc[:, i * 128:(i + 1) * 128]
        out_refs[nt_out][...] = jnp.dot(acc, ap_ref[...], preferred_element_type=_f32)

    grid = (n // blk,)
    in_specs = [
        pl.BlockSpec((2, nt_in, blk, 128), lambda i: (0, 0, i, 0)),
        pl.BlockSpec((2, blk, 16), lambda i: (0, i, 0)),
        pl.BlockSpec((heads, ch_in), lambda i: (0, 0)),
        pl.BlockSpec((heads, ch_in, d_out), lambda i: (0, 0, 0)),
        pl.BlockSpec((d_out, 128), lambda i: (0, 0)),
    ]
    out_specs = [pl.BlockSpec((blk, 128), lambda i: (i, 0)) for _ in range(nt_out + 1)]
    out_shape = [jax.ShapeDtypeStruct((n, 128), _f32) for _ in range(nt_out + 1)]
    return pl.pallas_call(body, grid=grid, in_specs=in_specs,
                          out_specs=out_specs, out_shape=out_shape)


def _make_tc3(n, heads, ch_in, ncls, blk):
    nt_in = heads * ch_in // 128

    def body(p_ref, dn_ref, b_ref, w_ref, bfc_ref, out_ref):
        p = p_ref[...]
        dn = dn_ref[...]
        b = b_ref[...]
        acc = jnp.zeros((blk, ncls), _f32) + bfc_ref[...]
        for h in range(heads):
            xh = _combine(p, dn, b, h, ch_in)
            acc = acc + jnp.dot(xh, w_ref[h], preferred_element_type=_f32)
        out_ref[...] = acc

    grid = (n // blk,)
    in_specs = [
        pl.BlockSpec((2, nt_in, blk, 128), lambda i: (0, 0, i, 0)),
        pl.BlockSpec((2, blk, 16), lambda i: (0, i, 0)),
        pl.BlockSpec((heads, ch_in), lambda i: (0, 0)),
        pl.BlockSpec((heads, ch_in, ncls), lambda i: (0, 0, 0)),
        pl.BlockSpec((1, ncls), lambda i: (0, 0)),
    ]
    out_specs = pl.BlockSpec((blk, ncls), lambda i: (i, 0))
    out_shape = jax.ShapeDtypeStruct((n, ncls), _f32)
    return pl.pallas_call(body, grid=grid, in_specs=in_specs,
                          out_specs=out_specs, out_shape=out_shape)


def _make_sca(n, e):
    epw = e // NW
    nblk = epw // BB
    npair = nblk // 2
    assert nblk % 2 == 0
    rpt, rpt_last = _row_split(n)
    mesh = plsc.VectorSubcoreMesh(core_axis_name="c", subcore_axis_name="s",
                                  num_cores=NC, num_subcores=NS)

    @functools.partial(
        pl.kernel,
        out_type=(jax.ShapeDtypeStruct((e, 16), _f32),
                  jax.ShapeDtypeStruct((NC, n, 16), _f32)),
        mesh=mesh,
        scratch_types=[
            pltpu.VMEM((2, BB), _i32),
            pltpu.VMEM((2, BB), _i32),
            pltpu.VMEM((2, BB, 128), _f32),
            pltpu.VMEM((2, BB, 128), _f32),
            pltpu.VMEM((2, BB, 16), _f32),
            pltpu.VMEM_SHARED((n, 16), _f32),
            pltpu.SemaphoreType.DMA,
            pltpu.SemaphoreType.DMA,
        ],
    )
    def sca(src_ref, dst_ref, asad_ref, zrow_ref, ex_ref, dpart_ref,
            src_v, dst_v, arows, brows, exbuf, den_acc, sem0, sem1):
        sems = (sem0, sem1)
        cid = lax.axis_index("c")
        sid = lax.axis_index("s")
        wid = sid * NC + cid
        base_e = wid * epw

        r0 = _mo8(sid * rpt)
        _tile_copy(lambda s: zrow_ref.at[pl.ds(0, s)],
                   lambda s: den_acc.at[pl.ds(r0, s)], sid, rpt, rpt_last)
        plsc.subcore_barrier()

        def fetch(slot, b):
            e0 = _mo8(base_e + b * BB)
            pltpu.sync_copy(src_ref.at[pl.ds(e0, BB)], src_v.at[slot])
            pltpu.sync_copy(dst_ref.at[pl.ds(e0, BB)], dst_v.at[slot])
            pltpu.async_copy(asad_ref.at[src_v.at[slot]], arows.at[slot],
                             sems[slot])
            pltpu.async_copy(asad_ref.at[dst_v.at[slot]], brows.at[slot],
                             sems[slot])

        def waitg(slot):
            pltpu.make_async_copy(asad_ref.at[src_v.at[slot]], arows.at[slot],
                                  sems[slot]).wait()
            pltpu.make_async_copy(asad_ref.at[dst_v.at[slot]], brows.at[slot],
                                  sems[slot]).wait()

        def process(slot, b):
            @plsc.parallel_loop(0, BB, unroll=4)
            def _(i):
                s = arows[slot, i, pl.ds(0, 16)] + brows[slot, i, pl.ds(16, 16)]
                s = jnp.where(s >= 0.0, s, SLOPE * s)
                exbuf[slot, i, :] = jnp.exp(s)

            e0 = _mo8(base_e + b * BB)
            pltpu.sync_copy(exbuf.at[slot], ex_ref.at[pl.ds(e0, BB)])
            pltpu.sync_copy(exbuf.at[slot], den_acc.at[dst_v.at[slot]],
                            add=True)

        fetch(0, 0)

        def pair(j, _):
            for k in (0, 1):
                b = 2 * j + k

                @pl.when(b + 1 < nblk)
                def _(k=k, b=b):
                    fetch(1 - k, b + 1)

                waitg(k)
                process(k, b)
            return 0

        lax.fori_loop(0, npair, pair, 0)
        plsc.subcore_barrier()

        _tile_copy(lambda s: den_acc.at[pl.ds(r0, s)],
                   lambda s: dpart_ref.at[cid, pl.ds(r0, s)],
                   sid, rpt, rpt_last)

    return sca


def _make_scb(n, e, heads, ch):
    epw = e // NW
    nblk = epw // BB
    npair = nblk // 2
    assert nblk % 2 == 0
    rpt, rpt_last = _row_split(n)
    hpt = 128 // ch
    nt = heads // hpt
    mesh = plsc.VectorSubcoreMesh(core_axis_name="c", subcore_axis_name="s",
                                  num_cores=NC, num_subcores=NS)

    @functools.partial(
        pl.kernel,
        out_type=jax.ShapeDtypeStruct((NC, nt, n, 128), _f32),
        mesh=mesh,
        scratch_types=[
            pltpu.VMEM((2, BB), _i32),
            pltpu.VMEM((2, BB), _i32),
            pltpu.VMEM((2, BB, 16), _f32),
            pltpu.VMEM((2, BB, 128), _f32),
            pltpu.VMEM_SHARED((n, 128), _f32),
            pltpu.SemaphoreType.DMA,
            pltpu.SemaphoreType.DMA,
        ],
    )
    def scb(src_ref, dst_ref, ex_ref, zrow_ref, *rest):
        h_refs = rest[:nt]
        part_ref = rest[nt]
        (src_v, dst_v, exb, grows, acc, sem0, sem1) = rest[nt + 1:]
        sems = (sem0, sem1)

        cid = lax.axis_index("c")
        sid = lax.axis_index("s")
        wid = sid * NC + cid
        base_e = wid * epw
        r0 = _mo8(sid * rpt)

        for t in range(nt):
            _tile_copy(lambda s: zrow_ref.at[pl.ds(0, s)],
                       lambda s: acc.at[pl.ds(r0, s)], sid, rpt, rpt_last)
            plsc.subcore_barrier()

            def fetch(slot, b, t=t):
                e0 = _mo8(base_e + b * BB)
                pltpu.sync_copy(src_ref.at[pl.ds(e0, BB)], src_v.at[slot])
                pltpu.sync_copy(dst_ref.at[pl.ds(e0, BB)], dst_v.at[slot])
                pltpu.sync_copy(ex_ref.at[pl.ds(e0, BB)], exb.at[slot])
                pltpu.async_copy(h_refs[t].at[src_v.at[slot]], grows.at[slot],
                                 sems[slot])

            def waitg(slot, t=t):
                pltpu.make_async_copy(h_refs[t].at[src_v.at[slot]],
                                      grows.at[slot], sems[slot]).wait()

            def process(slot, t=t):
                @plsc.parallel_loop(0, BB, unroll=4)
                def _(i):
                    exrow = exb[slot, i, :]
                    for k in range(hpt):
                        esp = _bcast_lane(exrow, t * hpt + k)
                        for cc in range(ch // 16):
                            sl = pl.ds(k * ch + cc * 16, 16)
                            grows[slot, i, sl] = grows[slot, i, sl] * esp

                pltpu.sync_copy(grows.at[slot], acc.at[dst_v.at[slot]],
                                add=True)

            fetch(0, 0)

            def pair(j, _, t=t):
                for k in (0, 1):
                    b = 2 * j + k

                    @pl.when(b + 1 < nblk)
                    def _(k=k, b=b):
                        fetch(1 - k, b + 1)

                    waitg(k)
                    process(k)
                return 0

            lax.fori_loop(0, npair, pair, 0)
            plsc.subcore_barrier()

            _tile_copy(lambda s: acc.at[pl.ds(r0, s)],
                       lambda s: part_ref.at[cid, t, pl.ds(r0, s)],
                       sid, rpt, rpt_last)
            plsc.subcore_barrier()

    return scb


def _apack(a_src, a_dst):
    heads, ch = a_src.shape
    eye = jnp.eye(heads, dtype=_f32)
    asrc = jnp.einsum("hc,hj->hcj", a_src, eye).reshape(heads * ch, heads)
    adst = jnp.einsum("hc,hj->hcj", a_dst, eye).reshape(heads * ch, heads)
    z8 = jnp.zeros((heads * ch, 16 - heads), _f32)
    zrest = jnp.zeros((heads * ch, 128 - 32), _f32)
    return jnp.concatenate([asrc, z8, adst, z8, zrest], axis=1)


def _gat_pipeline(n, e, f_in, heads, c1, c2, ncls, blk):
    tc1 = _make_tc1(n, f_in, heads, c1, blk)
    tc2 = _make_tc2(n, heads, c1, c2, blk)
    tc3 = _make_tc3(n, heads, c2, ncls, blk)
    sca = _make_sca(n, e)
    scb1 = _make_scb(n, e, heads, c1)
    scb2 = _make_scb(n, e, heads, c2)

    def run(x, ei, W1, a1s, a1d, b1, W2, a2s, a2d, b2, Wfc, bfc):
        ei = ei.astype(_i32)
        src = ei[0]
        dst = ei[1]
        zr = _row_split(n)[0]
        zrowA = jnp.zeros((zr, 16), _f32)
        zrowB = jnp.zeros((zr, 128), _f32)
        nt1 = heads * c1 // 128
        nt2 = heads * c2 // 128

        out1 = tc1(x, W1, _apack(a1s, a1d))
        ex1, dp1 = sca(src, dst, out1[nt1], zrowA)
        part1 = scb1(src, dst, ex1, zrowB, *out1[:nt1])

        out2 = tc2(part1, dp1, b1.reshape(heads, c1),
                   W2.reshape(heads, c1, heads * c2), _apack(a2s, a2d))
        ex2, dp2 = sca(src, dst, out2[nt2], zrowA)
        part2 = scb2(src, dst, ex2, zrowB, *out2[:nt2])

        logits = tc3(part2, dp2, b2.reshape(heads, c2),
                     Wfc.reshape(heads, c2, ncls), bfc.reshape(1, ncls))
        return logits

    return run


_pipeline = None


def kernel(x, edge_index, W1, a1_src, a1_dst, b1, W2, a2_src, a2_dst, b2, Wfc, bfc):
    global _pipeline
    if _pipeline is None:
        _pipeline = _gat_pipeline(N_NODES, N_EDGES, F_IN, HEADS, C1, C2,
                                  NUM_CLASS, blk=1000)
    return _pipeline(x, edge_index, W1, a1_src, a1_dst, b1, W2, a2_src,
                     a2_dst, b2, Wfc, bfc)


# speedup vs baseline: 1.3652x; 1.3652x over previous
import functools

import jax
import jax.numpy as jnp
from jax import lax
from jax.experimental import pallas as pl
from jax.experimental.pallas import tpu as pltpu
from jax.experimental.pallas import tpu_sc as plsc

N_NODES = 10000
N_EDGES = 320000
F_IN = 128
HEADS = 8
C1 = 128
C2 = 64
NUM_CLASS = 40
SLOPE = 0.1
EPS = 1e-16

NC = 2
NS = 16
NW = NC * NS
BB = 80

_f32 = jnp.float32
_i32 = jnp.int32


def _mo8(v):
    return pl.multiple_of(v, 8)


_GDN = lax.GatherDimensionNumbers(offset_dims=(), collapsed_slice_dims=(0,),
                                  start_index_map=(0,))


def _bcast_lane(vec, lane):
    idx = jnp.full((16, 1), lane, _i32)
    return lax.gather(vec, idx, _GDN, (1,),
                      mode=lax.GatherScatterMode.PROMISE_IN_BOUNDS)


def _row_split(n):
    main = ((n + NS - 1) // NS + 7) // 8 * 8
    last = n - (NS - 1) * main
    assert last > 0 and last % 8 == 0
    return main, last


def _tile_copy(src_fn, dst_fn, sid, main, last):
    @pl.when(sid < NS - 1)
    def _():
        pltpu.sync_copy(src_fn(main), dst_fn(main))

    @pl.when(sid == NS - 1)
    def _():
        pltpu.sync_copy(src_fn(last), dst_fn(last))


def _make_tc1(n, f_in, heads, ch, blk):
    d = heads * ch

    nt = d // 128

    def body(x_ref, w_ref, ap_ref, *out_refs):
        h = jnp.dot(x_ref[...], w_ref[...], preferred_element_type=_f32)
        for i in range(nt):
            out_refs[i][...] = h[:, i * 128:(i + 1) * 128]
        out_refs[nt][...] = jnp.dot(h, ap_ref[...], preferred_element_type=_f32)

    grid = (n // blk,)
    in_specs = [
        pl.BlockSpec((blk, f_in), lambda i: (i, 0)),
        pl.BlockSpec((f_in, d), lambda i: (0, 0)),
        pl.BlockSpec((d, 128), lambda i: (0, 0)),
    ]
    out_specs = [pl.BlockSpec((blk, 128), lambda i: (i, 0)) for _ in range(nt + 1)]
    out_shape = [jax.ShapeDtypeStruct((n, 128), _f32) for _ in range(nt + 1)]
    return pl.pallas_call(body, grid=grid, in_specs=in_specs,
                          out_specs=out_specs, out_shape=out_shape)


def _combine(p, dn, b, h, ch):
    hpt = 128 // ch
    t, off = h // hpt, (h % hpt) * ch
    rec = 1.0 / (dn[0] + dn[1] + EPS)
    ph = p[0, t][:, off:off + ch] + p[1, t][:, off:off + ch]
    xh = ph * rec[:, h][:, None] + b[h][None, :]
    return jnp.maximum(xh, 0.0)


def _make_tc2(n, heads, ch_in, ch_out, blk):
    d_out = heads * ch_out
    nt_in = heads * ch_in // 128
    nt_out = d_out // 128

    def body(p_ref, dn_ref, b_ref, w_ref, ap_ref, *out_refs):
        p = p_ref[...]
        dn = dn_ref[...]
        b = b_ref[...]
        acc = jnp.zeros((blk, d_out), _f32)
        for h in range(heads):
            xh = _combine(p, dn, b, h, ch_in)
            acc = acc + jnp.dot(xh, w_ref[h], preferred_element_type=_f32)
        for i in range(nt_out):
            out_refs[i][...] = acc[:, i * 128:(i + 1) * 128]
        out_refs[nt_out][...] = jnp.dot(acc, ap_ref[...], preferred_element_type=_f32)

    grid = (n // blk,)
    in_specs = [
        pl.BlockSpec((2, nt_in, blk, 128), lambda i: (0, 0, i, 0)),
        pl.BlockSpec((2, blk, 16), lambda i: (0, i, 0)),
        pl.BlockSpec((heads, ch_in), lambda i: (0, 0)),
        pl.BlockSpec((heads, ch_in, d_out), lambda i: (0, 0, 0)),
        pl.BlockSpec((d_out, 128), lambda i: (0, 0)),
    ]
    out_specs = [pl.BlockSpec((blk, 128), lambda i: (i, 0)) for _ in range(nt_out + 1)]
    out_shape = [jax.ShapeDtypeStruct((n, 128), _f32) for _ in range(nt_out + 1)]
    return pl.pallas_call(body, grid=grid, in_specs=in_specs,
                          out_specs=out_specs, out_shape=out_shape)


def _make_tc3(n, heads, ch_in, ncls, blk):
    nt_in = heads * ch_in // 128

    def body(p_ref, dn_ref, b_ref, w_ref, bfc_ref, out_ref):
        p = p_ref[...]
        dn = dn_ref[...]
        b = b_ref[...]
        acc = jnp.zeros((blk, ncls), _f32) + bfc_ref[...]
        for h in range(heads):
            xh = _combine(p, dn, b, h, ch_in)
            acc = acc + jnp.dot(xh, w_ref[h], preferred_element_type=_f32)
        out_ref[...] = acc

    grid = (n // blk,)
    in_specs = [
        pl.BlockSpec((2, nt_in, blk, 128), lambda i: (0, 0, i, 0)),
        pl.BlockSpec((2, blk, 16), lambda i: (0, i, 0)),
        pl.BlockSpec((heads, ch_in), lambda i: (0, 0)),
        pl.BlockSpec((heads, ch_in, ncls), lambda i: (0, 0, 0)),
        pl.BlockSpec((1, ncls), lambda i: (0, 0)),
    ]
    out_specs = pl.BlockSpec((blk, ncls), lambda i: (i, 0))
    out_shape = jax.ShapeDtypeStruct((n, ncls), _f32)
    return pl.pallas_call(body, grid=grid, in_specs=in_specs,
                          out_specs=out_specs, out_shape=out_shape)


def _make_sca(n, e):
    epw = e // NW
    nblk = epw // BB
    rpt, rpt_last = _row_split(n)
    mesh = plsc.VectorSubcoreMesh(core_axis_name="c", subcore_axis_name="s",
                                  num_cores=NC, num_subcores=NS)

    @functools.partial(
        pl.kernel,
        out_type=(jax.ShapeDtypeStruct((e * 16,), _f32),
                  jax.ShapeDtypeStruct((NC, n, 16), _f32)),
        mesh=mesh,
        scratch_types=[
            pltpu.VMEM((BB,), _i32),
            pltpu.VMEM((BB,), _i32),
            pltpu.VMEM((BB, 128), _f32),
            pltpu.VMEM((BB, 128), _f32),
            pltpu.VMEM((BB, 16), _f32),
            pltpu.VMEM((BB * 16,), _f32),
            pltpu.VMEM_SHARED((n, 16), _f32),
            pltpu.SemaphoreType.DMA,
            pltpu.SemaphoreType.DMA,
        ],
    )
    def sca(src_ref, dst_ref, asad_ref, zrow_ref, ex_ref, dpart_ref,
            src_v, dst_v, arows, brows, exbuf, exflat, den_acc, sem, sem2):
        cid = lax.axis_index("c")
        sid = lax.axis_index("s")
        wid = sid * NC + cid
        base_e = wid * epw

        r0 = _mo8(sid * rpt)
        _tile_copy(lambda s: zrow_ref.at[pl.ds(0, s)],
                   lambda s: den_acc.at[pl.ds(r0, s)], sid, rpt, rpt_last)
        plsc.subcore_barrier()

        def blk(b, _):
            e0 = _mo8(base_e + b * BB)
            pltpu.sync_copy(src_ref.at[pl.ds(e0, BB)], src_v)
            pltpu.sync_copy(dst_ref.at[pl.ds(e0, BB)], dst_v)
            cpa = pltpu.async_copy(asad_ref.at[src_v], arows, sem)
            cpb = pltpu.async_copy(asad_ref.at[dst_v], brows, sem2)
            cpa.wait()
            cpb.wait()

            def edge(i, _2):
                s = arows[i, pl.ds(0, 16)] + brows[i, pl.ds(16, 16)]
                s = jnp.where(s >= 0.0, s, SLOPE * s)
                ex = jnp.exp(s)
                exbuf[i, :] = ex
                exflat[pl.ds(i * 16, 16)] = ex
                return 0

            lax.fori_loop(0, BB, edge, 0)
            pltpu.sync_copy(exflat, ex_ref.at[pl.ds(_mo8(e0 * 16), BB * 16)])
            pltpu.sync_copy(exbuf, den_acc.at[dst_v], add=True)
            return 0

        lax.fori_loop(0, nblk, blk, 0)
        plsc.subcore_barrier()

        _tile_copy(lambda s: den_acc.at[pl.ds(r0, s)],
                   lambda s: dpart_ref.at[cid, pl.ds(r0, s)],
                   sid, rpt, rpt_last)

    return sca


def _make_scb(n, e, heads, ch):
    epw = e // NW
    nblk = epw // BB
    rpt, rpt_last = _row_split(n)
    hpt = 128 // ch
    nt = heads // hpt
    mesh = plsc.VectorSubcoreMesh(core_axis_name="c", subcore_axis_name="s",
                                  num_cores=NC, num_subcores=NS)

    @functools.partial(
        pl.kernel,
        out_type=jax.ShapeDtypeStruct((NC, nt, n, 128), _f32),
        mesh=mesh,
        scratch_types=[
            pltpu.VMEM((BB,), _i32),
            pltpu.VMEM((BB,), _i32),
            pltpu.VMEM((BB * 16,), _f32),
            pltpu.VMEM((BB, 128), _f32),
            pltpu.VMEM((BB,), _i32),
            pltpu.VMEM((BB,), _i32),
            pltpu.VMEM((BB * 16,), _f32),
            pltpu.VMEM((BB, 128), _f32),
            pltpu.VMEM_SHARED((n, 128), _f32),
            pltpu.SemaphoreType.DMA,
            pltpu.SemaphoreType.DMA,
        ],
    )
    def scb(src_ref, dst_ref, ex_ref, zrow_ref, *rest):
        h_refs = rest[:nt]
        part_ref = rest[nt]
        (src_v0, dst_v0, exflat0, grows0,
         src_v1, dst_v1, exflat1, grows1, acc, sem0, sem1) = rest[nt + 1:]

        cid = lax.axis_index("c")
        sid = lax.axis_index("s")
        wid = sid * NC + cid
        base_e = wid * epw
        r0 = _mo8(sid * rpt)

        for t in range(nt):
            _tile_copy(lambda s: zrow_ref.at[pl.ds(0, s)],
                       lambda s: acc.at[pl.ds(r0, s)], sid, rpt, rpt_last)
            plsc.subcore_barrier()

            def issue(bidx, src_v, dst_v, exflat, grows, sem, t=t):
                e0 = _mo8(base_e + bidx * BB)
                pltpu.sync_copy(src_ref.at[pl.ds(e0, BB)], src_v)
                pltpu.sync_copy(dst_ref.at[pl.ds(e0, BB)], dst_v)
                pltpu.sync_copy(ex_ref.at[pl.ds(_mo8(e0 * 16), BB * 16)], exflat)
                return pltpu.async_copy(h_refs[t].at[src_v], grows, sem)

            def consume(dst_v, exflat, grows, cp, t=t):
                cp.wait()

                def edge(i, _2):
                    exrow = exflat[pl.ds(i * 16, 16)]
                    for k in range(hpt):
                        esp = _bcast_lane(exrow, t * hpt + k)
                        for cc in range(ch // 16):
                            sl = pl.ds(k * ch + cc * 16, 16)
                            grows[i, sl] = grows[i, sl] * esp
                    return 0

                lax.fori_loop(0, BB, edge, 0)
                pltpu.sync_copy(grows, acc.at[dst_v], add=True)

            def blk2(b, _):
                cp0 = issue(2 * b, src_v0, dst_v0, exflat0, grows0, sem0)
                cp1 = issue(2 * b + 1, src_v1, dst_v1, exflat1, grows1, sem1)
                consume(dst_v0, exflat0, grows0, cp0)
                consume(dst_v1, exflat1, grows1, cp1)
                return 0

            lax.fori_loop(0, nblk // 2, blk2, 0)
            if nblk % 2:
                cpt = issue(nblk - 1, src_v0, dst_v0, exflat0, grows0, sem0)
                consume(dst_v0, exflat0, grows0, cpt)
            plsc.subcore_barrier()

            _tile_copy(lambda s: acc.at[pl.ds(r0, s)],
                       lambda s: part_ref.at[cid, t, pl.ds(r0, s)],
                       sid, rpt, rpt_last)
            plsc.subcore_barrier()

    return scb


def _apack(a_src, a_dst):
    heads, ch = a_src.shape
    eye = jnp.eye(heads, dtype=_f32)
    asrc = jnp.einsum("hc,hj->hcj", a_src, eye).reshape(heads * ch, heads)
    adst = jnp.einsum("hc,hj->hcj", a_dst, eye).reshape(heads * ch, heads)
    z8 = jnp.zeros((heads * ch, 16 - heads), _f32)
    zrest = jnp.zeros((heads * ch, 128 - 32), _f32)
    return jnp.concatenate([asrc, z8, adst, z8, zrest], axis=1)


def _gat_pipeline(n, e, f_in, heads, c1, c2, ncls, blk):
    tc1 = _make_tc1(n, f_in, heads, c1, blk)
    tc2 = _make_tc2(n, heads, c1, c2, blk)
    tc3 = _make_tc3(n, heads, c2, ncls, blk)
    sca = _make_sca(n, e)
    scb1 = _make_scb(n, e, heads, c1)
    scb2 = _make_scb(n, e, heads, c2)

    def run(x, ei, W1, a1s, a1d, b1, W2, a2s, a2d, b2, Wfc, bfc):
        ei = ei.astype(_i32)
        src = ei[0]
        dst = ei[1]
        zr = _row_split(n)[0]
        zrowA = jnp.zeros((zr, 16), _f32)
        zrowB = jnp.zeros((zr, 128), _f32)
        nt1 = heads * c1 // 128
        nt2 = heads * c2 // 128

        out1 = tc1(x, W1, _apack(a1s, a1d))
        ex1, dp1 = sca(src, dst, out1[nt1], zrowA)
        part1 = scb1(src, dst, ex1, zrowB, *out1[:nt1])

        out2 = tc2(part1, dp1, b1.reshape(heads, c1),
                   W2.reshape(heads, c1, heads * c2), _apack(a2s, a2d))
        ex2, dp2 = sca(src, dst, out2[nt2], zrowA)
        part2 = scb2(src, dst, ex2, zrowB, *out2[:nt2])

        logits = tc3(part2, dp2, b2.reshape(heads, c2),
                     Wfc.reshape(heads, c2, ncls), bfc.reshape(1, ncls))
        return logits

    return run


_pipeline = None


def kernel(x, edge_index, W1, a1_src, a1_dst, b1, W2, a2_src, a2_dst, b2, Wfc, bfc):
    global _pipeline
    if _pipeline is None:
        _pipeline = _gat_pipeline(N_NODES, N_EDGES, F_IN, HEADS, C1, C2,
                                  NUM_CLASS, blk=1000)
    return _pipeline(x, edge_index, W1, a1_src, a1_dst, b1, W2, a2_src,
                     a2_dst, b2, Wfc, bfc)
